# Initial kernel scaffold; baseline (speedup 1.0000x reference)
import jax
import jax.numpy as jnp
from jax.experimental import pallas as pl


def kernel(video_emb, audio_emb, y, view1_mem, view2_mem, positive_set, resmp_idx, rand_idx):
    raise NotImplementedError("write your pallas kernel here")


# speedup vs baseline: 9.2206x; 9.2206x over previous
import jax
import jax.numpy as jnp
from jax import lax
from jax.experimental import pallas as pl
from jax.experimental.pallas import tpu as pltpu
from jax.experimental.pallas import tpu_sc as plsc

_MEM = 100000
_DIM = 64
_BS = 1024
_POSK = 32
_NPOS = 8
_NNEG = 1024
_T = 0.07

_NC = 2
_NS = 16
_NW = _NC * _NS
_BPW = _BS // _NW
_NCHUNK = 4
_CH = _NNEG // _NCHUNK
_L = 16
_NSP = 1 + _NPOS

_C_V2A_POS = 0
_C_V2A_NEG = _L
_C_A2V_POS = _L + _NNEG
_C_A2V_NEG = 2 * _L + _NNEG
_OUTW = 2 * _L + 2 * _NNEG


def _norm_body(v_ref, a_ref, qv_ref, qa_ref):
    v = v_ref[...]
    a = a_ref[...]
    inv_t = 1.0 / _T
    qv_ref[...] = v * (lax.rsqrt(jnp.sum(v * v, axis=1, keepdims=True)) * inv_t)
    qa_ref[...] = a * (lax.rsqrt(jnp.sum(a * a, axis=1, keepdims=True)) * inv_t)


def _dot_group(rows_ref, row_of_j, qc, iota):
    res = jnp.zeros((_L,), jnp.float32)
    for j in range(_L):
        r = row_of_j(j)
        p = rows_ref[r, pl.ds(0, _L)] * qc[0]
        p = p + rows_ref[r, pl.ds(_L, _L)] * qc[1]
        p = p + rows_ref[r, pl.ds(2 * _L, _L)] * qc[2]
        p = p + rows_ref[r, pl.ds(3 * _L, _L)] * qc[3]
        res = jnp.where(iota == j, jnp.sum(p), res)
    return res


def _sc_body(qv_hbm, qa_hbm, y_hbm, v1_hbm, v2_hbm, pset_hbm, resmp_hbm,
             rand_hbm, out_hbm,
             y_v, posset, posflat, refflat, posidx, spidx, qv_v, qa_v,
             sp1, sp2, resmp_v, rand_v, negidx, rows1, rows2, outrow,
             sem0, sem1):
    wid = lax.axis_index("s") * _NC + lax.axis_index("c")
    base = wid * _BPW
    iota = lax.iota(jnp.int32, _L)

    pltpu.sync_copy(y_hbm.at[pl.ds(base, _BPW)], y_v)
    pltpu.sync_copy(qv_hbm.at[pl.ds(base, _BPW)], qv_v)
    pltpu.sync_copy(qa_hbm.at[pl.ds(base, _BPW)], qa_v)
    pltpu.sync_copy(resmp_hbm.at[pl.ds(base * _NPOS, _BPW * _NPOS)], resmp_v)

    pltpu.async_copy(pset_hbm.at[y_v], posset, sem0).wait()

    def _ref_body(r, _):
        h0 = posset[r, pl.ds(0, _L)]
        h1 = posset[r, pl.ds(_L, _L)]
        posflat[pl.ds(r * _POSK, _L)] = h0
        posflat[pl.ds(r * _POSK + _L, _L)] = h1
        refflat[pl.ds(r * _POSK, _L)] = h0 - iota
        refflat[pl.ds(r * _POSK + _L, _L)] = h1 - (iota + _L)
        return 0
    lax.fori_loop(0, _BPW, _ref_body, 0)

    for g in range(_BPW * _NPOS // _L):
        brow = 2 * g + lax.shift_right_logical(iota, 3)
        rvec = resmp_v[pl.ds(g * _L, _L)]
        posidx[pl.ds(g * _L, _L)] = plsc.load_gather(
            posflat, [brow * _POSK + rvec])

    for g in range(_BPW * _NSP // _L):
        t = g * _L + iota
        b16 = lax.div(t, _NSP)
        i16 = t - b16 * _NSP
        yb = plsc.load_gather(y_v, [b16])
        pp = plsc.load_gather(posidx, [jnp.maximum(b16 * _NPOS + i16 - 1, 0)])
        spidx[pl.ds(g * _L, _L)] = jnp.where(i16 == 0, yb, pp)

    cp1 = pltpu.async_copy(v1_hbm.at[spidx], sp1, sem0)
    cp2 = pltpu.async_copy(v2_hbm.at[spidx], sp2, sem1)
    cp1.wait()
    cp2.wait()


    def _b_body(b, _):
        bg = base + b
        qv = tuple(qv_v[b, pl.ds(c * _L, _L)] for c in range(_DIM // _L))
        qa = tuple(qa_v[b, pl.ds(c * _L, _L)] for c in range(_DIM // _L))

        sp_of_j = lambda j: b * _NSP + min(j, _NSP - 1)
        outrow[pl.ds(_C_V2A_POS, _L)] = _dot_group(sp2, sp_of_j, qv, iota)
        outrow[pl.ds(_C_A2V_POS, _L)] = _dot_group(sp1, sp_of_j, qa, iota)

        def _chunk_body(c, _):
            pltpu.sync_copy(rand_hbm.at[bg, pl.ds(c * _CH, _CH)], rand_v)

            def _grp_body(g, _):
                r16 = rand_v[pl.ds(g * _L, _L)]
                acc = r16
                for k in range(_POSK):
                    refk = plsc.load_gather(
                        refflat, [jnp.full((_L,), b * _POSK + k, jnp.int32)])
                    acc = acc + (r16 >= refk).astype(jnp.int32)
                negidx[pl.ds(g * _L, _L)] = acc
                return 0
            lax.fori_loop(0, _CH // _L, _grp_body, 0)

            g1 = pltpu.async_copy(v1_hbm.at[negidx], rows1, sem0)
            g2 = pltpu.async_copy(v2_hbm.at[negidx], rows2, sem1)
            g1.wait()
            g2.wait()

            def _dot_body(g, _):
                row_of_j = lambda j: g * _L + j
                outrow[pl.ds(_C_V2A_NEG + c * _CH + g * _L, _L)] = _dot_group(
                    rows2, row_of_j, qv, iota)
                outrow[pl.ds(_C_A2V_NEG + c * _CH + g * _L, _L)] = _dot_group(
                    rows1, row_of_j, qa, iota)
                return 0
            lax.fori_loop(0, _CH // _L, _dot_body, 0)
            return 0
        lax.fori_loop(0, _NCHUNK, _chunk_body, 0)

        pltpu.sync_copy(outrow, out_hbm.at[bg])
        return 0
    lax.fori_loop(0, _BPW, _b_body, 0)


@jax.jit
def kernel(video_emb, audio_emb, y, view1_mem, view2_mem, positive_set,
           resmp_idx, rand_idx):
    qv, qa = pl.pallas_call(
        _norm_body,
        out_shape=[
            jax.ShapeDtypeStruct((_BS, _DIM), jnp.float32),
            jax.ShapeDtypeStruct((_BS, _DIM), jnp.float32),
        ],
    )(video_emb, audio_emb)

    mesh = plsc.VectorSubcoreMesh(core_axis_name="c", subcore_axis_name="s")
    sc = pl.kernel(
        _sc_body,
        out_type=jax.ShapeDtypeStruct((_BS, _OUTW), jnp.float32),
        mesh=mesh,
        compiler_params=pltpu.CompilerParams(needs_layout_passes=False, use_tc_tiling_on_sc=False),
        scratch_types=[
            pltpu.VMEM((_BPW,), jnp.int32),
            pltpu.VMEM((_BPW, _POSK), jnp.int32),
            pltpu.VMEM((_BPW * _POSK,), jnp.int32),
            pltpu.VMEM((_BPW * _POSK,), jnp.int32),
            pltpu.VMEM((_BPW * _NPOS,), jnp.int32),
            pltpu.VMEM((_BPW * _NSP,), jnp.int32),
            pltpu.VMEM((_BPW, _DIM), jnp.float32),
            pltpu.VMEM((_BPW, _DIM), jnp.float32),
            pltpu.VMEM((_BPW * _NSP, _DIM), jnp.float32),
            pltpu.VMEM((_BPW * _NSP, _DIM), jnp.float32),
            pltpu.VMEM((_BPW * _NPOS,), jnp.int32),
            pltpu.VMEM((_CH,), jnp.int32),
            pltpu.VMEM((_CH,), jnp.int32),
            pltpu.VMEM((_CH, _DIM), jnp.float32),
            pltpu.VMEM((_CH, _DIM), jnp.float32),
            pltpu.VMEM((_OUTW,), jnp.float32),
            pltpu.SemaphoreType.DMA,
            pltpu.SemaphoreType.DMA,
        ],
    )
    padded = sc(qv, qa, y, view1_mem, view2_mem, positive_set,
                resmp_idx.reshape(-1), rand_idx)
    return jnp.concatenate([
        padded[:, _C_V2A_POS:_C_V2A_POS + _NSP],
        padded[:, _C_V2A_NEG:_C_V2A_NEG + _NNEG],
        padded[:, _C_A2V_POS:_C_A2V_POS + _NSP],
        padded[:, _C_A2V_NEG:_C_A2V_NEG + _NNEG],
    ], axis=1)


# speedup vs baseline: 11.9152x; 1.2922x over previous
import jax
import jax.numpy as jnp
from jax import lax
from jax.experimental import pallas as pl
from jax.experimental.pallas import tpu as pltpu
from jax.experimental.pallas import tpu_sc as plsc

_MEM = 100000
_DIM = 64
_BS = 1024
_POSK = 32
_NPOS = 8
_NNEG = 1024
_T = 0.07

_NC = 2
_NS = 16
_NW = _NC * _NS
_BPW = _BS // _NW
_NCHUNK = 4
_CH = _NNEG // _NCHUNK
_L = 16
_NSP = 1 + _NPOS

_C_V2A_POS = 0
_C_V2A_NEG = _L
_C_A2V_POS = _L + _NNEG
_C_A2V_NEG = 2 * _L + _NNEG
_OUTW = 2 * _L + 2 * _NNEG


def _norm_body(v_ref, a_ref, qv_ref, qa_ref):
    v = v_ref[...]
    a = a_ref[...]
    inv_t = 1.0 / _T
    qv_ref[...] = v * (lax.rsqrt(jnp.sum(v * v, axis=1, keepdims=True)) * inv_t)
    qa_ref[...] = a * (lax.rsqrt(jnp.sum(a * a, axis=1, keepdims=True)) * inv_t)


def _dot_group(rows_ref, row_of_j, qc, iota):
    res = jnp.zeros((_L,), jnp.float32)
    for j in range(_L):
        r = row_of_j(j)
        p = rows_ref[r, pl.ds(0, _L)] * qc[0]
        p = p + rows_ref[r, pl.ds(_L, _L)] * qc[1]
        p = p + rows_ref[r, pl.ds(2 * _L, _L)] * qc[2]
        p = p + rows_ref[r, pl.ds(3 * _L, _L)] * qc[3]
        res = jnp.where(iota == j, jnp.sum(p), res)
    return res


def _sc_body(qv_hbm, qa_hbm, y_hbm, v1_hbm, v2_hbm, pset_hbm, resmp_hbm,
             rand_hbm, out_hbm,
             y_v, posset, posflat, refflat, posidx, spidx, qv_v, qa_v,
             sp1, sp2, resmp_v, rand_v, negidx0, negidx1,
             rows1a, rows1b, rows2a, rows2b, outrow,
             sem0, sem1, gsem0a, gsem0b, gsem1a, gsem1b):
    wid = lax.axis_index("s") * _NC + lax.axis_index("c")
    base = wid * _BPW
    iota = lax.iota(jnp.int32, _L)

    pltpu.sync_copy(y_hbm.at[pl.ds(base, _BPW)], y_v)
    pltpu.sync_copy(qv_hbm.at[pl.ds(base, _BPW)], qv_v)
    pltpu.sync_copy(qa_hbm.at[pl.ds(base, _BPW)], qa_v)
    pltpu.sync_copy(resmp_hbm.at[pl.ds(base * _NPOS, _BPW * _NPOS)], resmp_v)

    pltpu.async_copy(pset_hbm.at[y_v], posset, sem0).wait()

    def _ref_body(r, _):
        h0 = posset[r, pl.ds(0, _L)]
        h1 = posset[r, pl.ds(_L, _L)]
        posflat[pl.ds(r * _POSK, _L)] = h0
        posflat[pl.ds(r * _POSK + _L, _L)] = h1
        refflat[pl.ds(r * _POSK, _L)] = h0 - iota
        refflat[pl.ds(r * _POSK + _L, _L)] = h1 - (iota + _L)
        return 0
    lax.fori_loop(0, _BPW, _ref_body, 0)

    for g in range(_BPW * _NPOS // _L):
        brow = 2 * g + lax.shift_right_logical(iota, 3)
        rvec = resmp_v[pl.ds(g * _L, _L)]
        posidx[pl.ds(g * _L, _L)] = plsc.load_gather(
            posflat, [brow * _POSK + rvec])

    for g in range(_BPW * _NSP // _L):
        t = g * _L + iota
        b16 = lax.div(t, _NSP)
        i16 = t - b16 * _NSP
        yb = plsc.load_gather(y_v, [b16])
        pp = plsc.load_gather(posidx, [jnp.maximum(b16 * _NPOS + i16 - 1, 0)])
        spidx[pl.ds(g * _L, _L)] = jnp.where(i16 == 0, yb, pp)

    cp1 = pltpu.async_copy(v1_hbm.at[spidx], sp1, sem0)
    cp2 = pltpu.async_copy(v2_hbm.at[spidx], sp2, sem1)
    cp1.wait()
    cp2.wait()


    negbuf = (negidx0, negidx1)
    rbuf1 = (rows1a, rows1b)
    rbuf2 = (rows2a, rows2b)
    gsems = ((gsem0a, gsem0b), (gsem1a, gsem1b))

    def _b_body(b, _):
        bg = base + b
        qv = tuple(qv_v[b, pl.ds(c * _L, _L)] for c in range(_DIM // _L))
        qa = tuple(qa_v[b, pl.ds(c * _L, _L)] for c in range(_DIM // _L))

        prev = None
        for c in range(_NCHUNK + 1):
            p = c % 2
            if c < _NCHUNK:
                pltpu.sync_copy(rand_hbm.at[bg, pl.ds(c * _CH, _CH)], rand_v)
                nb = negbuf[p]

                def _grp_body(g, _):
                    r16 = rand_v[pl.ds(g * _L, _L)]
                    acc = r16
                    for k in range(_POSK):
                        refk = plsc.load_gather(
                            refflat,
                            [jnp.full((_L,), b * _POSK + k, jnp.int32)])
                        acc = acc + (r16 >= refk).astype(jnp.int32)
                    nb[pl.ds(g * _L, _L)] = acc
                    return 0
                lax.fori_loop(0, _CH // _L, _grp_body, 0)

                h1 = pltpu.async_copy(v1_hbm.at[nb], rbuf1[p], gsems[0][p])
                h2 = pltpu.async_copy(v2_hbm.at[nb], rbuf2[p], gsems[1][p])

            if c == 0:
                sp_of_j = lambda j: b * _NSP + min(j, _NSP - 1)
                outrow[pl.ds(_C_V2A_POS, _L)] = _dot_group(
                    sp2, sp_of_j, qv, iota)
                outrow[pl.ds(_C_A2V_POS, _L)] = _dot_group(
                    sp1, sp_of_j, qa, iota)
            else:
                prev[0].wait()
                prev[1].wait()
                cd = c - 1
                pd = cd % 2
                r1d, r2d = rbuf1[pd], rbuf2[pd]

                def _dot_body(g, _):
                    row_of_j = lambda j: g * _L + j
                    outrow[pl.ds(_C_V2A_NEG + cd * _CH + g * _L, _L)] = (
                        _dot_group(r2d, row_of_j, qv, iota))
                    outrow[pl.ds(_C_A2V_NEG + cd * _CH + g * _L, _L)] = (
                        _dot_group(r1d, row_of_j, qa, iota))
                    return 0
                lax.fori_loop(0, _CH // _L, _dot_body, 0)

            if c < _NCHUNK:
                prev = (h1, h2)

        pltpu.sync_copy(outrow, out_hbm.at[bg])
        return 0
    lax.fori_loop(0, _BPW, _b_body, 0)


@jax.jit
def kernel(video_emb, audio_emb, y, view1_mem, view2_mem, positive_set,
           resmp_idx, rand_idx):
    qv, qa = pl.pallas_call(
        _norm_body,
        out_shape=[
            jax.ShapeDtypeStruct((_BS, _DIM), jnp.float32),
            jax.ShapeDtypeStruct((_BS, _DIM), jnp.float32),
        ],
    )(video_emb, audio_emb)

    mesh = plsc.VectorSubcoreMesh(core_axis_name="c", subcore_axis_name="s")
    sc = pl.kernel(
        _sc_body,
        out_type=jax.ShapeDtypeStruct((_BS, _OUTW), jnp.float32),
        mesh=mesh,
        compiler_params=pltpu.CompilerParams(needs_layout_passes=False, use_tc_tiling_on_sc=False),
        scratch_types=[
            pltpu.VMEM((_BPW,), jnp.int32),
            pltpu.VMEM((_BPW, _POSK), jnp.int32),
            pltpu.VMEM((_BPW * _POSK,), jnp.int32),
            pltpu.VMEM((_BPW * _POSK,), jnp.int32),
            pltpu.VMEM((_BPW * _NPOS,), jnp.int32),
            pltpu.VMEM((_BPW * _NSP,), jnp.int32),
            pltpu.VMEM((_BPW, _DIM), jnp.float32),
            pltpu.VMEM((_BPW, _DIM), jnp.float32),
            pltpu.VMEM((_BPW * _NSP, _DIM), jnp.float32),
            pltpu.VMEM((_BPW * _NSP, _DIM), jnp.float32),
            pltpu.VMEM((_BPW * _NPOS,), jnp.int32),
            pltpu.VMEM((_CH,), jnp.int32),
            pltpu.VMEM((_CH,), jnp.int32),
            pltpu.VMEM((_CH,), jnp.int32),
            pltpu.VMEM((_CH, _DIM), jnp.float32),
            pltpu.VMEM((_CH, _DIM), jnp.float32),
            pltpu.VMEM((_CH, _DIM), jnp.float32),
            pltpu.VMEM((_CH, _DIM), jnp.float32),
            pltpu.VMEM((_OUTW,), jnp.float32),
            pltpu.SemaphoreType.DMA,
            pltpu.SemaphoreType.DMA,
            pltpu.SemaphoreType.DMA,
            pltpu.SemaphoreType.DMA,
            pltpu.SemaphoreType.DMA,
            pltpu.SemaphoreType.DMA,
        ],
    )
    padded = sc(qv, qa, y, view1_mem, view2_mem, positive_set,
                resmp_idx.reshape(-1), rand_idx)
    return jnp.concatenate([
        padded[:, _C_V2A_POS:_C_V2A_POS + _NSP],
        padded[:, _C_V2A_NEG:_C_V2A_NEG + _NNEG],
        padded[:, _C_A2V_POS:_C_A2V_POS + _NSP],
        padded[:, _C_A2V_NEG:_C_A2V_NEG + _NNEG],
    ], axis=1)


# speedup vs baseline: 12.5187x; 1.0506x over previous
import jax
import jax.numpy as jnp
from jax import lax
from jax.experimental import pallas as pl
from jax.experimental.pallas import tpu as pltpu
from jax.experimental.pallas import tpu_sc as plsc

_MEM = 100000
_DIM = 64
_BS = 1024
_POSK = 32
_NPOS = 8
_NNEG = 1024
_T = 0.07

_NC = 2
_NS = 16
_NW = _NC * _NS
_BPW = _BS // _NW
_NCHUNK = 4
_CH = _NNEG // _NCHUNK
_L = 16
_NSP = 1 + _NPOS

_C_V2A_POS = 0
_C_V2A_NEG = _L
_C_A2V_POS = _L + _NNEG
_C_A2V_NEG = 2 * _L + _NNEG
_OUTW = 2 * _L + 2 * _NNEG


def _norm_body(v_ref, a_ref, qv_ref, qa_ref):
    v = v_ref[...]
    a = a_ref[...]
    inv_t = 1.0 / _T
    qv_ref[...] = v * (lax.rsqrt(jnp.sum(v * v, axis=1, keepdims=True)) * inv_t)
    qa_ref[...] = a * (lax.rsqrt(jnp.sum(a * a, axis=1, keepdims=True)) * inv_t)


def _dot_group(rows_ref, row_of_j, qc, iota):
    res = jnp.zeros((_L,), jnp.float32)
    for j in range(_L):
        r = row_of_j(j)
        a0, b0 = plsc.unpack(rows_ref[r, pl.ds(0, 2 * _L)],
                             format=plsc.PackFormat.INTERLEAVED)
        a1, b1 = plsc.unpack(rows_ref[r, pl.ds(2 * _L, 2 * _L)],
                             format=plsc.PackFormat.INTERLEAVED)
        p = a0 * qc[0] + b0 * qc[1] + a1 * qc[2] + b1 * qc[3]
        res = jnp.where(iota == j, jnp.sum(p), res)
    return res


def _sc_body(qv_hbm, qa_hbm, y_hbm, v1_hbm, v2_hbm, pset_hbm, resmp_hbm,
             rand_hbm, out_hbm,
             y_v, posset, posflat, refflat, posidx, spidx, qv_v, qa_v,
             sp1, sp2, resmp_v, rand_v, negidx0, negidx1,
             rows1a, rows1b, rows2a, rows2b, outrow,
             sem0, sem1, gsem0a, gsem0b, gsem1a, gsem1b):
    wid = lax.axis_index("s") * _NC + lax.axis_index("c")
    base = wid * _BPW
    iota = lax.iota(jnp.int32, _L)

    pltpu.sync_copy(y_hbm.at[pl.ds(base, _BPW)], y_v)
    pltpu.sync_copy(qv_hbm.at[pl.ds(base, _BPW)], qv_v)
    pltpu.sync_copy(qa_hbm.at[pl.ds(base, _BPW)], qa_v)
    pltpu.sync_copy(resmp_hbm.at[pl.ds(base * _NPOS, _BPW * _NPOS)], resmp_v)

    pltpu.async_copy(pset_hbm.at[y_v], posset, sem0).wait()

    def _ref_body(r, _):
        h0 = posset[r, pl.ds(0, _L)]
        h1 = posset[r, pl.ds(_L, _L)]
        posflat[pl.ds(r * _POSK, _L)] = h0
        posflat[pl.ds(r * _POSK + _L, _L)] = h1
        refflat[pl.ds(r * _POSK, _L)] = h0 - iota
        refflat[pl.ds(r * _POSK + _L, _L)] = h1 - (iota + _L)
        return 0
    lax.fori_loop(0, _BPW, _ref_body, 0)

    for g in range(_BPW * _NPOS // _L):
        brow = 2 * g + lax.shift_right_logical(iota, 3)
        rvec = resmp_v[pl.ds(g * _L, _L)]
        posidx[pl.ds(g * _L, _L)] = plsc.load_gather(
            posflat, [brow * _POSK + rvec])

    for g in range(_BPW * _NSP // _L):
        t = g * _L + iota
        b16 = lax.div(t, _NSP)
        i16 = t - b16 * _NSP
        yb = plsc.load_gather(y_v, [b16])
        pp = plsc.load_gather(posidx, [jnp.maximum(b16 * _NPOS + i16 - 1, 0)])
        spidx[pl.ds(g * _L, _L)] = jnp.where(i16 == 0, yb, pp)

    cp1 = pltpu.async_copy(v1_hbm.at[spidx], sp1, sem0)
    cp2 = pltpu.async_copy(v2_hbm.at[spidx], sp2, sem1)
    cp1.wait()
    cp2.wait()


    negbuf = (negidx0, negidx1)
    rbuf1 = (rows1a, rows1b)
    rbuf2 = (rows2a, rows2b)
    gsems = ((gsem0a, gsem0b), (gsem1a, gsem1b))

    def _b_body(b, _):
        bg = base + b
        qv = tuple(qv_v[b, pl.ds(c * _L, _L)] for c in range(_DIM // _L))
        qa = tuple(qa_v[b, pl.ds(c * _L, _L)] for c in range(_DIM // _L))

        prev = None
        for c in range(_NCHUNK + 1):
            p = c % 2
            if c < _NCHUNK:
                pltpu.sync_copy(rand_hbm.at[bg, pl.ds(c * _CH, _CH)], rand_v)
                nb = negbuf[p]

                refA = refflat[pl.ds(b * _POSK, _L)]
                refB = refflat[pl.ds(b * _POSK + _L, _L)]

                def _grp_body(g, _):
                    r16 = rand_v[pl.ds(g * _L, _L)]
                    acc = r16
                    for k in range(_L):
                        acc = acc + (r16 >= refA[k]).astype(jnp.int32)
                        acc = acc + (r16 >= refB[k]).astype(jnp.int32)
                    nb[pl.ds(g * _L, _L)] = acc
                    return 0
                lax.fori_loop(0, _CH // _L, _grp_body, 0)

                h1 = pltpu.async_copy(v1_hbm.at[nb], rbuf1[p], gsems[0][p])
                h2 = pltpu.async_copy(v2_hbm.at[nb], rbuf2[p], gsems[1][p])

            if c == 0:
                sp_of_j = lambda j: b * _NSP + min(j, _NSP - 1)
                outrow[pl.ds(_C_V2A_POS, _L)] = _dot_group(
                    sp2, sp_of_j, qv, iota)
                outrow[pl.ds(_C_A2V_POS, _L)] = _dot_group(
                    sp1, sp_of_j, qa, iota)
            else:
                prev[0].wait()
                prev[1].wait()
                cd = c - 1
                pd = cd % 2
                r1d, r2d = rbuf1[pd], rbuf2[pd]

                def _dot_body(g, _):
                    row_of_j = lambda j: g * _L + j
                    outrow[pl.ds(_C_V2A_NEG + cd * _CH + g * _L, _L)] = (
                        _dot_group(r2d, row_of_j, qv, iota))
                    outrow[pl.ds(_C_A2V_NEG + cd * _CH + g * _L, _L)] = (
                        _dot_group(r1d, row_of_j, qa, iota))
                    return 0
                lax.fori_loop(0, _CH // _L, _dot_body, 0)

            if c < _NCHUNK:
                prev = (h1, h2)

        pltpu.sync_copy(outrow, out_hbm.at[bg])
        return 0
    lax.fori_loop(0, _BPW, _b_body, 0)


@jax.jit
def kernel(video_emb, audio_emb, y, view1_mem, view2_mem, positive_set,
           resmp_idx, rand_idx):
    qv, qa = pl.pallas_call(
        _norm_body,
        out_shape=[
            jax.ShapeDtypeStruct((_BS, _DIM), jnp.float32),
            jax.ShapeDtypeStruct((_BS, _DIM), jnp.float32),
        ],
    )(video_emb, audio_emb)

    mesh = plsc.VectorSubcoreMesh(core_axis_name="c", subcore_axis_name="s")
    sc = pl.kernel(
        _sc_body,
        out_type=jax.ShapeDtypeStruct((_BS, _OUTW), jnp.float32),
        mesh=mesh,
        compiler_params=pltpu.CompilerParams(needs_layout_passes=False, use_tc_tiling_on_sc=False),
        scratch_types=[
            pltpu.VMEM((_BPW,), jnp.int32),
            pltpu.VMEM((_BPW, _POSK), jnp.int32),
            pltpu.VMEM((_BPW * _POSK,), jnp.int32),
            pltpu.VMEM((_BPW * _POSK,), jnp.int32),
            pltpu.VMEM((_BPW * _NPOS,), jnp.int32),
            pltpu.VMEM((_BPW * _NSP,), jnp.int32),
            pltpu.VMEM((_BPW, _DIM), jnp.float32),
            pltpu.VMEM((_BPW, _DIM), jnp.float32),
            pltpu.VMEM((_BPW * _NSP, _DIM), jnp.bfloat16),
            pltpu.VMEM((_BPW * _NSP, _DIM), jnp.bfloat16),
            pltpu.VMEM((_BPW * _NPOS,), jnp.int32),
            pltpu.VMEM((_CH,), jnp.int32),
            pltpu.VMEM((_CH,), jnp.int32),
            pltpu.VMEM((_CH,), jnp.int32),
            pltpu.VMEM((_CH, _DIM), jnp.bfloat16),
            pltpu.VMEM((_CH, _DIM), jnp.bfloat16),
            pltpu.VMEM((_CH, _DIM), jnp.bfloat16),
            pltpu.VMEM((_CH, _DIM), jnp.bfloat16),
            pltpu.VMEM((_OUTW,), jnp.float32),
            pltpu.SemaphoreType.DMA,
            pltpu.SemaphoreType.DMA,
            pltpu.SemaphoreType.DMA,
            pltpu.SemaphoreType.DMA,
            pltpu.SemaphoreType.DMA,
            pltpu.SemaphoreType.DMA,
        ],
    )
    b1 = view1_mem.astype(jnp.bfloat16)
    b2 = view2_mem.astype(jnp.bfloat16)
    qv = qv.reshape(_BS, 2, _L, 2).transpose(0, 1, 3, 2).reshape(_BS, _DIM)
    qa = qa.reshape(_BS, 2, _L, 2).transpose(0, 1, 3, 2).reshape(_BS, _DIM)
    padded = sc(qv, qa, y, b1, b2, positive_set,
                resmp_idx.reshape(-1), rand_idx)
    return jnp.concatenate([
        padded[:, _C_V2A_POS:_C_V2A_POS + _NSP],
        padded[:, _C_V2A_NEG:_C_V2A_NEG + _NNEG],
        padded[:, _C_A2V_POS:_C_A2V_POS + _NSP],
        padded[:, _C_A2V_NEG:_C_A2V_NEG + _NNEG],
    ], axis=1)


# speedup vs baseline: 14.1396x; 1.1295x over previous
import jax
import jax.numpy as jnp
from jax import lax
from jax.experimental import pallas as pl
from jax.experimental.pallas import tpu as pltpu
from jax.experimental.pallas import tpu_sc as plsc

_MEM = 100000
_DIM = 64
_BS = 1024
_POSK = 32
_NPOS = 8
_NNEG = 1024
_T = 0.07

_NC = 2
_NS = 16
_NW = _NC * _NS
_BPW = _BS // _NW
_NCHUNK = 4
_CH = _NNEG // _NCHUNK
_L = 16
_NSP = 1 + _NPOS

_C_V2A_POS = 0
_C_V2A_NEG = _L
_C_A2V_POS = _L + _NNEG
_C_A2V_NEG = 2 * _L + _NNEG
_OUTW = 2 * _L + 2 * _NNEG


def _norm_body(v_ref, a_ref, qv_ref, qa_ref):
    v = v_ref[...]
    a = a_ref[...]
    inv_t = 1.0 / _T
    qv_ref[...] = v * (lax.rsqrt(jnp.sum(v * v, axis=1, keepdims=True)) * inv_t)
    qa_ref[...] = a * (lax.rsqrt(jnp.sum(a * a, axis=1, keepdims=True)) * inv_t)


def _dot_group(rows_ref, row_of_j, qc, iota):
    res = jnp.zeros((_L,), jnp.float32)
    for j in range(_L):
        r = row_of_j(j)
        ps = (rows_ref[r, pl.ds(0, 2 * _L)] * qc[0]
              + rows_ref[r, pl.ds(2 * _L, 2 * _L)] * qc[1])
        a, b = plsc.unpack(ps, format=plsc.PackFormat.INTERLEAVED)
        res = jnp.where(iota == j, jnp.sum(a + b), res)
    return res


def _sc_body(qv_hbm, qa_hbm, y_hbm, v1_hbm, v2_hbm, pset_hbm, resmp_hbm,
             rand_hbm, out_hbm,
             y_v, posset, posflat, refflat, posidx, spidx, qv_v, qa_v,
             sp1, sp2, resmp_v, rand_v, negidx0, negidx1,
             rows1a, rows1b, rows2a, rows2b, outrow,
             sem0, sem1, gsem0a, gsem0b, gsem1a, gsem1b):
    wid = lax.axis_index("s") * _NC + lax.axis_index("c")
    base = wid * _BPW
    iota = lax.iota(jnp.int32, _L)

    pltpu.sync_copy(y_hbm.at[pl.ds(base, _BPW)], y_v)
    pltpu.sync_copy(qv_hbm.at[pl.ds(base, _BPW)], qv_v)
    pltpu.sync_copy(qa_hbm.at[pl.ds(base, _BPW)], qa_v)
    pltpu.sync_copy(resmp_hbm.at[pl.ds(base * _NPOS, _BPW * _NPOS)], resmp_v)

    pltpu.async_copy(pset_hbm.at[y_v], posset, sem0).wait()

    def _ref_body(r, _):
        h0 = posset[r, pl.ds(0, _L)]
        h1 = posset[r, pl.ds(_L, _L)]
        posflat[pl.ds(r * _POSK, _L)] = h0
        posflat[pl.ds(r * _POSK + _L, _L)] = h1
        refflat[pl.ds(r * _POSK, _L)] = h0 - iota
        refflat[pl.ds(r * _POSK + _L, _L)] = h1 - (iota + _L)
        return 0
    lax.fori_loop(0, _BPW, _ref_body, 0)

    for g in range(_BPW * _NPOS // _L):
        brow = 2 * g + lax.shift_right_logical(iota, 3)
        rvec = resmp_v[pl.ds(g * _L, _L)]
        posidx[pl.ds(g * _L, _L)] = plsc.load_gather(
            posflat, [brow * _POSK + rvec])

    for g in range(_BPW * _NSP // _L):
        t = g * _L + iota
        b16 = lax.div(t, _NSP)
        i16 = t - b16 * _NSP
        yb = plsc.load_gather(y_v, [b16])
        pp = plsc.load_gather(posidx, [jnp.maximum(b16 * _NPOS + i16 - 1, 0)])
        spidx[pl.ds(g * _L, _L)] = jnp.where(i16 == 0, yb, pp)

    cp1 = pltpu.async_copy(v1_hbm.at[spidx], sp1, sem0)
    cp2 = pltpu.async_copy(v2_hbm.at[spidx], sp2, sem1)
    cp1.wait()
    cp2.wait()


    negbuf = (negidx0, negidx1)
    rbuf1 = (rows1a, rows1b)
    rbuf2 = (rows2a, rows2b)
    gsems = ((gsem0a, gsem0b), (gsem1a, gsem1b))

    def _b_body(b, _):
        bg = base + b
        qv = tuple(qv_v[b, pl.ds(c * 2 * _L, 2 * _L)] for c in range(2))
        qa = tuple(qa_v[b, pl.ds(c * 2 * _L, 2 * _L)] for c in range(2))

        prev = None
        for c in range(_NCHUNK + 1):
            p = c % 2
            if c == 0:
                pltpu.sync_copy(rand_hbm.at[bg], rand_v)
            if c < _NCHUNK:
                nb = negbuf[p]

                refA = refflat[pl.ds(b * _POSK, _L)]
                refB = refflat[pl.ds(b * _POSK + _L, _L)]

                def _grp_body(g, _):
                    r16 = rand_v[pl.ds(c * _CH + g * _L, _L)]
                    acc = r16
                    for k in range(_L):
                        acc = acc + (r16 >= refA[k]).astype(jnp.int32)
                        acc = acc + (r16 >= refB[k]).astype(jnp.int32)
                    nb[pl.ds(g * _L, _L)] = acc
                    return 0
                lax.fori_loop(0, _CH // _L, _grp_body, 0)

                h1 = pltpu.async_copy(v1_hbm.at[nb], rbuf1[p], gsems[0][p])
                h2 = pltpu.async_copy(v2_hbm.at[nb], rbuf2[p], gsems[1][p])

            if c == 0:
                sp_of_j = lambda j: b * _NSP + min(j, _NSP - 1)
                outrow[pl.ds(_C_V2A_POS, _L)] = _dot_group(
                    sp2, sp_of_j, qv, iota)
                outrow[pl.ds(_C_A2V_POS, _L)] = _dot_group(
                    sp1, sp_of_j, qa, iota)
            else:
                prev[0].wait()
                prev[1].wait()
                cd = c - 1
                pd = cd % 2
                r1d, r2d = rbuf1[pd], rbuf2[pd]

                def _dot_body(g, _):
                    row_of_j = lambda j: g * _L + j
                    outrow[pl.ds(_C_V2A_NEG + cd * _CH + g * _L, _L)] = (
                        _dot_group(r2d, row_of_j, qv, iota))
                    outrow[pl.ds(_C_A2V_NEG + cd * _CH + g * _L, _L)] = (
                        _dot_group(r1d, row_of_j, qa, iota))
                    return 0
                lax.fori_loop(0, _CH // _L, _dot_body, 0)

            if c < _NCHUNK:
                prev = (h1, h2)

        pltpu.sync_copy(outrow, out_hbm.at[bg])
        return 0
    lax.fori_loop(0, _BPW, _b_body, 0)


@jax.jit
def kernel(video_emb, audio_emb, y, view1_mem, view2_mem, positive_set,
           resmp_idx, rand_idx):
    qv, qa = pl.pallas_call(
        _norm_body,
        out_shape=[
            jax.ShapeDtypeStruct((_BS, _DIM), jnp.float32),
            jax.ShapeDtypeStruct((_BS, _DIM), jnp.float32),
        ],
    )(video_emb, audio_emb)

    mesh = plsc.VectorSubcoreMesh(core_axis_name="c", subcore_axis_name="s")
    sc = pl.kernel(
        _sc_body,
        out_type=jax.ShapeDtypeStruct((_BS, _OUTW), jnp.float32),
        mesh=mesh,
        compiler_params=pltpu.CompilerParams(needs_layout_passes=False, use_tc_tiling_on_sc=False),
        scratch_types=[
            pltpu.VMEM((_BPW,), jnp.int32),
            pltpu.VMEM((_BPW, _POSK), jnp.int32),
            pltpu.VMEM((_BPW * _POSK,), jnp.int32),
            pltpu.VMEM((_BPW * _POSK,), jnp.int32),
            pltpu.VMEM((_BPW * _NPOS,), jnp.int32),
            pltpu.VMEM((_BPW * _NSP,), jnp.int32),
            pltpu.VMEM((_BPW, _DIM), jnp.bfloat16),
            pltpu.VMEM((_BPW, _DIM), jnp.bfloat16),
            pltpu.VMEM((_BPW * _NSP, _DIM), jnp.bfloat16),
            pltpu.VMEM((_BPW * _NSP, _DIM), jnp.bfloat16),
            pltpu.VMEM((_BPW * _NPOS,), jnp.int32),
            pltpu.VMEM((_NNEG,), jnp.int32),
            pltpu.VMEM((_CH,), jnp.int32),
            pltpu.VMEM((_CH,), jnp.int32),
            pltpu.VMEM((_CH, _DIM), jnp.bfloat16),
            pltpu.VMEM((_CH, _DIM), jnp.bfloat16),
            pltpu.VMEM((_CH, _DIM), jnp.bfloat16),
            pltpu.VMEM((_CH, _DIM), jnp.bfloat16),
            pltpu.VMEM((_OUTW,), jnp.float32),
            pltpu.SemaphoreType.DMA,
            pltpu.SemaphoreType.DMA,
            pltpu.SemaphoreType.DMA,
            pltpu.SemaphoreType.DMA,
            pltpu.SemaphoreType.DMA,
            pltpu.SemaphoreType.DMA,
        ],
    )
    b1 = view1_mem.astype(jnp.bfloat16)
    b2 = view2_mem.astype(jnp.bfloat16)
    padded = sc(qv.astype(jnp.bfloat16), qa.astype(jnp.bfloat16), y, b1, b2,
                positive_set, resmp_idx.reshape(-1), rand_idx)
    return jnp.concatenate([
        padded[:, _C_V2A_POS:_C_V2A_POS + _NSP],
        padded[:, _C_V2A_NEG:_C_V2A_NEG + _NNEG],
        padded[:, _C_A2V_POS:_C_A2V_POS + _NSP],
        padded[:, _C_A2V_NEG:_C_A2V_NEG + _NNEG],
    ], axis=1)
